# Initial kernel scaffold; baseline (speedup 1.0000x reference)
import jax
import jax.numpy as jnp
from jax.experimental import pallas as pl


def kernel(x, edge_index, batch, W1, b1, W2, b2, gamma, beta, Wlin, blin):
    raise NotImplementedError("write your pallas kernel here")


# speedup vs baseline: 13.4443x; 13.4443x over previous
import functools

import jax
import jax.numpy as jnp
from jax import lax
from jax.experimental import pallas as pl
from jax.experimental.pallas import tpu as pltpu
from jax.experimental.pallas import tpu_sc as plsc

NNODES = 10000
NP = 10112
NEDGES = 320000
D = 128
G = 64
NC, NS = 2, 16
NW = NC * NS
EPT = NEDGES // NW
CH = 128
NCHUNK = 79
EPT_PAD = NCHUNK * CH
RPT = NP // NS

_mesh = plsc.VectorSubcoreMesh(
    core_axis_name="c", subcore_axis_name="s", num_cores=NC, num_subcores=NS
)


def _deg_body(dstp, ones_r, zeros_r, out, dst_idx, ones_v, acc):
    c = lax.axis_index("c")
    s = lax.axis_index("s")
    wid = c * NS + s
    pltpu.sync_copy(zeros_r, acc.at[pl.ds(s * RPT, RPT)])
    pltpu.sync_copy(ones_r, ones_v)
    pltpu.sync_copy(dstp.at[wid], dst_idx)
    plsc.subcore_barrier()

    def body(j, carry):
        pltpu.sync_copy(ones_v, acc.at[dst_idx.at[j]], add=True)
        return carry

    lax.fori_loop(0, NCHUNK, body, 0)
    plsc.subcore_barrier()
    pltpu.sync_copy(acc.at[pl.ds(s * RPT, RPT)],
                    out.at[pl.ds(c * NP + s * RPT, RPT)])


def _agg_body(g, srcp, dstp, out, src_idx, dst_idx, rows, acc, sem):
    c = lax.axis_index("c")
    s = lax.axis_index("s")
    wid = c * NS + s
    pltpu.sync_copy(g.at[pl.ds(s * RPT, RPT)], acc.at[pl.ds(s * RPT, RPT)])
    pltpu.sync_copy(srcp.at[wid], src_idx)
    pltpu.sync_copy(dstp.at[wid], dst_idx)
    plsc.subcore_barrier()

    def body(j, carry):
        pltpu.async_copy(g.at[src_idx.at[j]], rows, sem).wait()
        pltpu.sync_copy(rows, acc.at[dst_idx.at[j]], add=True)
        return carry

    lax.fori_loop(0, NCHUNK, body, 0)
    plsc.subcore_barrier()
    pltpu.sync_copy(acc.at[pl.ds(s * RPT, RPT)],
                    out.at[pl.ds(c * NP + s * RPT, RPT)])


def _build_deg_kernel(interpret=False):
    return pl.kernel(
        _deg_body,
        out_type=jax.ShapeDtypeStruct((NC * NP, D), jnp.float32),
        mesh=_mesh,
        scratch_types=[
            pltpu.VMEM((NCHUNK, CH), jnp.int32),
            pltpu.VMEM((CH, D), jnp.float32),
            pltpu.VMEM_SHARED((NP, D), jnp.float32),
        ],
        interpret=interpret,
    )


def _build_agg_kernel(interpret=False):
    return pl.kernel(
        _agg_body,
        out_type=jax.ShapeDtypeStruct((NC * NP, D), jnp.float32),
        mesh=_mesh,
        scratch_types=[
            pltpu.VMEM((NCHUNK, CH), jnp.int32),
            pltpu.VMEM((NCHUNK, CH), jnp.int32),
            pltpu.VMEM((CH, D), jnp.float32),
            pltpu.VMEM_SHARED((NP, D), jnp.float32),
            pltpu.SemaphoreType.DMA,
        ],
        interpret=interpret,
    )


_deg_kernel = _build_deg_kernel()
_agg_kernel = _build_agg_kernel()


def _prep_body(x_ref, degp_ref, w1_ref, g_ref, dis_ref):
    deg = degp_ref[0:NP, 0:1] + degp_ref[NP:2 * NP, 0:1] + 1.0
    dis = lax.rsqrt(deg)
    y = jnp.dot(x_ref[...], w1_ref[...], preferred_element_type=jnp.float32)
    g_ref[...] = y * dis
    dis_ref[...] = dis


_prep = pl.pallas_call(
    _prep_body,
    out_shape=[
        jax.ShapeDtypeStruct((NP, D), jnp.float32),
        jax.ShapeDtypeStruct((NP, 1), jnp.float32),
    ],
)


def _mid_body(p_ref, g_ref, dis_ref, b_ref, gamma_ref, beta_ref, w2_ref,
              g2_ref):
    dis = dis_ref[...]
    sarr = (p_ref[0:NP, :] + p_ref[NP:2 * NP, :] - g_ref[...]) * dis + b_ref[...]
    sv = sarr[0:NNODES, :]
    mean = jnp.mean(sv, axis=0, keepdims=True)
    var = jnp.mean(sv * sv, axis=0, keepdims=True) - mean * mean
    h = jnp.maximum(
        (sarr - mean) * lax.rsqrt(var + 1e-5) * gamma_ref[...] + beta_ref[...],
        0.0)
    g2_ref[...] = jnp.dot(
        h, w2_ref[...], preferred_element_type=jnp.float32) * dis


_mid = pl.pallas_call(
    _mid_body,
    out_shape=[jax.ShapeDtypeStruct((NP, D), jnp.float32)],
)


def _final_body(q_ref, g2_ref, dis_ref, b_ref, gamma_ref, beta_ref, batch_ref,
                wlin_ref, blin_ref, out_ref):
    dis = dis_ref[...]
    sarr = (q_ref[0:NP, :] + q_ref[NP:2 * NP, :] - g2_ref[...]) * dis + b_ref[...]
    sv = sarr[0:NNODES, :]
    mean = jnp.mean(sv, axis=0, keepdims=True)
    var = jnp.mean(sv * sv, axis=0, keepdims=True) - mean * mean
    h = jnp.maximum(
        (sarr - mean) * lax.rsqrt(var + 1e-5) * gamma_ref[...] + beta_ref[...],
        0.0)
    hv = h[0:NNODES, :]
    gids = lax.broadcasted_iota(jnp.int32, (G, NNODES), 0)
    onehot_t = (gids == batch_ref[...]).astype(jnp.float32)
    sums = jnp.dot(onehot_t, hv, preferred_element_type=jnp.float32)
    counts = jnp.sum(onehot_t, axis=1, keepdims=True)
    pooled = sums / jnp.maximum(counts, 1.0)
    out_ref[...] = jnp.dot(
        pooled, wlin_ref[...], preferred_element_type=jnp.float32) + blin_ref[...]


_final = pl.pallas_call(
    _final_body,
    out_shape=[jax.ShapeDtypeStruct((G, D), jnp.float32)],
)


def kernel(x, edge_index, batch, W1, b1, W2, b2, gamma, beta, Wlin, blin):
    f32 = jnp.float32
    src = edge_index[0].reshape(NW, EPT)
    dst = edge_index[1].reshape(NW, EPT)
    pad = EPT_PAD - EPT
    srcp = jnp.pad(src, ((0, 0), (0, pad))).reshape(NW, NCHUNK, CH)
    dstp = jnp.pad(dst, ((0, 0), (0, pad)),
                   constant_values=NNODES).reshape(NW, NCHUNK, CH)
    ones_r = jnp.ones((CH, D), f32)
    zeros_r = jnp.zeros((RPT, D), f32)

    deg_p = _deg_kernel(dstp, ones_r, zeros_r)
    xpad = jnp.pad(x, ((0, NP - NNODES), (0, 0)))
    g1, dis = _prep(xpad, deg_p, W1)
    p = _agg_kernel(g1, srcp, dstp)
    (g2,) = _mid(p, g1, dis, b1.reshape(1, D), gamma.reshape(1, D),
                 beta.reshape(1, D), W2)
    q = _agg_kernel(g2, srcp, dstp)
    (out,) = _final(q, g2, dis, b2.reshape(1, D), gamma.reshape(1, D),
                    beta.reshape(1, D), batch.reshape(1, NNODES), Wlin,
                    blin.reshape(1, D))
    return out
